# baseline (speedup 1.0000x reference)
import functools

import jax
import jax.numpy as jnp
from jax import lax
from jax.experimental import pallas as pl
from jax.experimental.pallas import tpu as pltpu
from jax.experimental.pallas import tpu_sc as plsc

_D = 32
_R = 16384
_C = 50
_CP = 128

_info = plsc.get_sparse_core_info()
_NC, _NS = _info.num_cores, _info.num_subcores
_NW = _NC * _NS
_R_PER_W = _R // _NW
_CR = 16
_CHUNK = _CR * _C
_N = _R_PER_W // _CR


@functools.partial(
    pl.kernel,
    out_type=jax.ShapeDtypeStruct((_R, _C, _D), jnp.float32),
    mesh=plsc.VectorSubcoreMesh(core_axis_name="c", subcore_axis_name="s"),
    scratch_types=[
        pltpu.VMEM((_CR, _CP), jnp.int32),
        pltpu.VMEM((_CHUNK,), jnp.int32),
        pltpu.VMEM((_CHUNK, _D), jnp.float32),
        pltpu.SemaphoreType.DMA,
    ],
    compiler_params=pltpu.CompilerParams(use_tc_tiling_on_sc=False),
)
def _emb_gather(x_hbm, table_hbm, out_hbm, cidx, fidx, rows, sem):
    wid = lax.axis_index("s") * _NC + lax.axis_index("c")
    row0 = wid * _R_PER_W

    def body(g, carry):
        r0 = row0 + g * _CR
        pltpu.sync_copy(x_hbm.at[pl.ds(r0, _CR), :], cidx)
        for r in range(_CR):
            for k in (0, 16, 32, 34):
                fidx[pl.ds(r * _C + k, 16)] = cidx[r, pl.ds(k, 16)]
        pltpu.async_copy(table_hbm.at[fidx], rows, sem).wait()
        for r in range(_CR):
            pltpu.sync_copy(rows.at[pl.ds(r * _C, _C), :], out_hbm.at[r0 + r])
        return carry

    lax.fori_loop(0, _N, body, 0)


def kernel(x, table):
    xp = jnp.pad(x.astype(jnp.int32), ((0, 0), (0, _CP - _C)))
    return _emb_gather(xp, table)


# speedup vs baseline: 1.0222x; 1.0222x over previous
import functools

import jax
import jax.numpy as jnp
from jax import lax
from jax.experimental import pallas as pl
from jax.experimental.pallas import tpu as pltpu
from jax.experimental.pallas import tpu_sc as plsc

_D = 32
_R = 16384
_C = 50
_B = _R * _C

_info = plsc.get_sparse_core_info()
_NC, _NS = _info.num_cores, _info.num_subcores
_NW = _NC * _NS
_R_PER_W = _R // _NW
_CR = 16
_CHUNK = _CR * _C
_N = _R_PER_W // _CR
_B_PER_W = _B // _NW

_mesh = plsc.VectorSubcoreMesh(core_axis_name="c", subcore_axis_name="s")


@functools.partial(
    pl.kernel,
    out_type=jax.ShapeDtypeStruct((_B,), jnp.int32),
    mesh=_mesh,
    scratch_types=[
        pltpu.VMEM((_CR, _C), jnp.int32),
        pltpu.VMEM((_CHUNK,), jnp.int32),
    ],
    compiler_params=pltpu.CompilerParams(use_tc_tiling_on_sc=True),
)
def _x_repack(x_hbm, x2_hbm, cidx, fidx):
    wid = lax.axis_index("s") * _NC + lax.axis_index("c")
    row0 = wid * _R_PER_W

    def body(g, carry):
        r0 = row0 + g * _CR
        pltpu.sync_copy(x_hbm.at[pl.ds(r0, _CR), :], cidx)
        for r in range(_CR):
            for k in (0, 16, 32, 34):
                fidx[pl.ds(r * _C + k, 16)] = cidx[r, pl.ds(k, 16)]
        pltpu.sync_copy(fidx, x2_hbm.at[pl.ds(r0 * _C, _CHUNK)])
        return carry

    lax.fori_loop(0, _N, body, 0)


@functools.partial(
    pl.kernel,
    out_type=jax.ShapeDtypeStruct((_R, _C, _D), jnp.float32),
    mesh=_mesh,
    scratch_types=[
        pltpu.VMEM((_B_PER_W,), jnp.int32),
        pltpu.VMEM((_CHUNK, _D), jnp.float32),
        pltpu.SemaphoreType.DMA,
    ],
    compiler_params=pltpu.CompilerParams(use_tc_tiling_on_sc=False),
)
def _emb_gather(x2_hbm, table_hbm, out_hbm, idx_all, rows, sem):
    wid = lax.axis_index("s") * _NC + lax.axis_index("c")
    row0 = wid * _R_PER_W
    pltpu.sync_copy(x2_hbm.at[pl.ds(row0 * _C, _B_PER_W)], idx_all)

    def body(g, carry):
        pltpu.async_copy(
            table_hbm.at[idx_all.at[pl.ds(g * _CHUNK, _CHUNK)]], rows, sem
        ).wait()
        r0 = row0 + g * _CR
        for r in range(_CR):
            pltpu.sync_copy(rows.at[pl.ds(r * _C, _C), :], out_hbm.at[r0 + r])
        return carry

    lax.fori_loop(0, _N, body, 0)


def kernel(x, table):
    x2 = _x_repack(x.astype(jnp.int32))
    return _emb_gather(x2, table)
